# baseline (speedup 1.0000x reference)
import functools

import jax
import jax.numpy as jnp
from jax import lax
from jax.experimental import pallas as pl
from jax.experimental.pallas import tpu as pltpu
from jax.experimental.pallas import tpu_sc as plsc

N = 10000
E = 160000
D = 256
NUM_GRAPHS = 16
MAX_DEG = 10
NUM_REL = 4
NUM_BLOCKS = 2
N_OUT = 128

NC = 2
NS = 16
NW = NC * NS
L = 16

N_PAD = 10240
R4 = NUM_REL * N_PAD
BK = 320
NB = R4 // BK
E_PAD = 163840
SLICE = E_PAD // NW
SP_CAP = 170000
KE = 4096
PAD_KEY = 0xFFFF

_SC_PARAMS = pltpu.CompilerParams(needs_layout_passes=False)


def _mesh():
    return plsc.VectorSubcoreMesh(core_axis_name="c", subcore_axis_name="s")


def _wid():
    return lax.axis_index("s") * NC + lax.axis_index("c")


def _hist_body(ed_hbm, hist_hbm, ebuf, hacc):
    wid = _wid()
    zero16 = jnp.zeros((L,), jnp.int32)

    for i in range(144 // L):
        hacc[pl.ds(i * L, L)] = zero16
    pltpu.sync_copy(ed_hbm.at[pl.ds(wid * SLICE, SLICE)], ebuf)

    def hgrp(g, _):
        mv = ebuf[pl.ds(g * L, L)]
        kv = mv & 0xFFFF
        bv = jnp.minimum(kv // BK, NB)
        rc, lastm = plsc.scan_count(bv)
        plsc.addupdate_scatter(hacc, [bv], rc, mask=lastm)
        return 0

    lax.fori_loop(0, SLICE // L, hgrp, 0)
    pltpu.sync_copy(hacc, hist_hbm.at[wid])


def _hist_call(edges):
    kfn = pl.kernel(
        _hist_body,
        out_type=jax.ShapeDtypeStruct((NW, 144), jnp.int32),
        mesh=_mesh(),
        scratch_types=[
            pltpu.VMEM((SLICE,), jnp.int32),
            pltpu.VMEM((144,), jnp.int32),
        ],
        compiler_params=_SC_PARAMS,
    )
    return kfn(edges)


def _cnt_body(sed_hbm, rng_hbm, cnt_hbm, rbuf, ebuf, cacc):
    wid = _wid()
    kpt = R4 // NW
    lo = wid * kpt
    iota = lax.iota(jnp.int32, L)
    pltpu.sync_copy(rng_hbm.at[wid], rbuf)
    rv = rbuf[pl.ds(0, L)]
    zero16 = jnp.zeros((L,), jnp.int32)
    for i in range((kpt + L) // L):
        cacc[pl.ds(i * L, L)] = zero16

    for sub in range(NUM_REL):
        est = pl.multiple_of(rv[8 + 2 * sub], 16)
        ecnt = rv[9 + 2 * sub]
        nch = (ecnt + KE - 1) // KE

        def chunk_body(ch, _):
            off0 = ch * KE
            pltpu.sync_copy(sed_hbm.at[pl.ds(est + off0, KE)],
                            ebuf.at[pl.ds(0, KE)])
            rem_c = ecnt - off0
            ng = jnp.minimum((rem_c + L - 1) // L, KE // L)

            def cgrp(g, _):
                mv = ebuf[pl.ds(g * L, L)]
                kv = mv & 0xFFFF
                valid = iota < (rem_c - g * L)
                lidx = jnp.where(valid, kv - lo, kpt)
                rc, lastm = plsc.scan_count(lidx, mask=valid)
                plsc.addupdate_scatter(cacc, [lidx], rc, mask=lastm)
                return 0

            lax.fori_loop(0, ng, cgrp, 0)
            return 0

        lax.fori_loop(0, nch, chunk_body, 0)
    pltpu.sync_copy(cacc.at[pl.ds(0, kpt)], cnt_hbm.at[pl.ds(lo, kpt)])


def _cnt_call(sedges, ranges):
    kfn = pl.kernel(
        _cnt_body,
        out_type=jax.ShapeDtypeStruct((R4,), jnp.int32),
        mesh=_mesh(),
        scratch_types=[
            pltpu.VMEM((L,), jnp.int32),
            pltpu.VMEM((KE + L,), jnp.int32),
            pltpu.VMEM((R4 // NW + L,), jnp.int32),
        ],
        compiler_params=_SC_PARAMS,
    )
    return kfn(sedges, ranges)


def _scatter_body(ed_hbm, start_hbm, out_hbm, ebuf, ctr, drain, sem):
    wid = _wid()
    pltpu.sync_copy(ed_hbm.at[pl.ds(wid * SLICE, SLICE)], ebuf)
    pltpu.sync_copy(start_hbm.at[wid], ctr)

    def grp(g, _):
        mv = ebuf[pl.ds(g * L, L)]
        kv = mv & 0xFFFF
        bv = jnp.minimum(kv // BK, NB)
        rc, lastm = plsc.scan_count(bv)
        base = plsc.load_gather(ctr, [bv])
        posv = base + rc - 1
        plsc.addupdate_scatter(ctr, [bv], rc, mask=lastm)
        pltpu.async_copy(ebuf.at[pl.ds(g * L, L)], out_hbm.at[posv], sem)
        return 0

    lax.fori_loop(0, SLICE // L, grp, 0)

    def dr(g, _):
        pltpu.make_async_copy(ed_hbm.at[pl.ds(0, L)], drain, sem).wait()
        return 0

    lax.fori_loop(0, SLICE // L, dr, 0)


def _scatter_call(edges, start_t):
    kfn = pl.kernel(
        _scatter_body,
        out_type=jax.ShapeDtypeStruct((SP_CAP,), jnp.int32),
        mesh=_mesh(),
        scratch_types=[
            pltpu.VMEM((SLICE,), jnp.int32),
            pltpu.VMEM((144,), jnp.int32),
            pltpu.VMEM((L,), jnp.int32),
            pltpu.SemaphoreType.DMA,
        ],
        compiler_params=_SC_PARAMS,
    )
    return kfn(edges, start_t)


def _rows_body(sed_hbm, rng_hbm, h_hbm, out_hbm, rbuf, ebuf, stg0, stg1, acc,
               sem0, sem1, *, mf):
    wid = _wid()
    pltpu.sync_copy(rng_hbm.at[wid], rbuf)
    rv = rbuf[pl.ds(0, L)]
    iota = lax.iota(jnp.int32, L)
    zero16 = jnp.zeros((L,), jnp.float32)

    n_rounds = 1 if mf else NUM_REL
    n_sub = NUM_REL if mf else 1
    for rnd in range(n_rounds):
        def zrow(i, _):
            for kk in range(D // L):
                acc[i, pl.ds(kk * L, L)] = zero16
            return 0

        lax.fori_loop(0, BK, zrow, 0)

        for sub in range(n_sub):
            base_w = (8 + 2 * sub) if mf else (2 * rnd)
            est = pl.multiple_of(rv[base_w], 16)
            ecnt = rv[base_w + 1]
            nch = (ecnt + KE - 1) // KE

            def lane_meta(rem_c, g):
                off = g * L
                mv = ebuf[pl.ds(off, L)]
                kv = mv & 0xFFFF
                sv = mv >> 16
                valid = iota < (rem_c - off)
                sv = jnp.where(valid, sv, 0)
                if mf:
                    lidx = (kv >> 2) - wid * BK
                else:
                    lidx = kv - (rnd * NW + wid) * BK
                lidx = jnp.where(valid, lidx, BK)
                return sv, lidx

            def gather(rem_c, g, stg, sem):
                sv, _ = lane_meta(rem_c, g)
                pltpu.async_copy(h_hbm.at[sv], stg, sem)

            def process(rem_c, g, stg, sem):
                _, lidx = lane_meta(rem_c, g)
                pltpu.make_async_copy(
                    h_hbm.at[pl.ds(0, L), :], stg, sem).wait()
                for i in range(L):
                    li = lidx[i]

                    @plsc.parallel_loop(0, D, step=L, unroll=8)
                    def _(kk):
                        plsc.addupdate(acc.at[li, pl.ds(kk, L)],
                                       stg[i, pl.ds(kk, L)])

            def chunk_body(ch, _):
                off0 = ch * KE
                pltpu.sync_copy(sed_hbm.at[pl.ds(est + off0, KE)],
                                ebuf.at[pl.ds(0, KE)])
                rem_c = ecnt - off0
                ng = jnp.minimum((rem_c + L - 1) // L, KE // L)
                ngp = (ng + 1) // 2
                gather(rem_c, 0, stg0, sem0)

                def pair(p, _):
                    g0 = 2 * p
                    gather(rem_c, g0 + 1, stg1, sem1)
                    process(rem_c, g0, stg0, sem0)
                    gather(rem_c, g0 + 2, stg0, sem0)
                    process(rem_c, g0 + 1, stg1, sem1)
                    return 0

                lax.fori_loop(0, ngp, pair, 0)
                pltpu.make_async_copy(
                    h_hbm.at[pl.ds(0, L), :], stg0, sem0).wait()
                return 0

            lax.fori_loop(0, nch, chunk_body, 0)

        obase = (wid if mf else rnd * NW + wid) * BK
        pltpu.sync_copy(acc.at[pl.ds(0, BK), :], out_hbm.at[pl.ds(obase, BK), :])


def _rows_call(sedges, ranges, h, mf):
    rows = N_PAD if mf else R4
    kfn = pl.kernel(
        functools.partial(_rows_body, mf=mf),
        out_type=jax.ShapeDtypeStruct((rows, D), jnp.float32),
        mesh=_mesh(),
        scratch_types=[
            pltpu.VMEM((L,), jnp.int32),
            pltpu.VMEM((KE + 2 * L,), jnp.int32),
            pltpu.VMEM((L, D), jnp.float32),
            pltpu.VMEM((L, D), jnp.float32),
            pltpu.VMEM((BK + L, D), jnp.float32),
            pltpu.SemaphoreType.DMA,
            pltpu.SemaphoreType.DMA,
        ],
        compiler_params=_SC_PARAMS,
    )
    return kfn(sedges, ranges, h)


def _ceil16(x):
    return ((x + 15) // 16) * 16


BT = 256
GRID = N_PAD // BT


def _emb_body(x_ref, w_ref, b_ref, o_ref):
    o_ref[...] = jnp.maximum(
        jnp.dot(x_ref[...], w_ref[...],
                preferred_element_type=jnp.float32) + b_ref[...], 0.0)


def _emb_call(xp, emb_W, emb_b):
    return pl.pallas_call(
        _emb_body,
        grid=(GRID,),
        in_specs=[
            pl.BlockSpec((BT, D), lambda i: (i, 0)),
            pl.BlockSpec((D, D), lambda i: (0, 0)),
            pl.BlockSpec((1, D), lambda i: (0, 0)),
        ],
        out_specs=pl.BlockSpec((BT, D), lambda i: (i, 0)),
        out_shape=jax.ShapeDtypeStruct((N_PAD, D), jnp.float32),
    )(xp, emb_W, emb_b.reshape(1, D))


def _rgcn_body(a_ref, buf_ref, inv_ref, root_ref, w_ref, b_ref, o_ref):
    a = a_ref[...].astype(jnp.bfloat16)
    acc = jnp.dot(a, root_ref[...], preferred_element_type=jnp.float32)
    for r in range(NUM_REL):
        mean = (buf_ref[:, r, :] * inv_ref[:, r, :]).astype(jnp.bfloat16)
        acc = acc + jnp.dot(mean, w_ref[r], preferred_element_type=jnp.float32)
    o_ref[...] = jnp.maximum(acc + b_ref[...], 0.0)


def _rgcn_call(a, buf4, inv4c, root, W, b):
    return pl.pallas_call(
        _rgcn_body,
        grid=(GRID,),
        in_specs=[
            pl.BlockSpec((BT, D), lambda i: (i, 0)),
            pl.BlockSpec((BT, NUM_REL, D), lambda i: (i, 0, 0)),
            pl.BlockSpec((BT, NUM_REL, 1), lambda i: (i, 0, 0)),
            pl.BlockSpec((D, D), lambda i: (0, 0)),
            pl.BlockSpec((NUM_REL, D, D), lambda i: (0, 0, 0)),
            pl.BlockSpec((1, D), lambda i: (0, 0)),
        ],
        out_specs=pl.BlockSpec((BT, D), lambda i: (i, 0)),
        out_shape=jax.ShapeDtypeStruct((N_PAD, D), jnp.float32),
    )(a, buf4, inv4c, root, W, b.reshape(1, D))


def _mf_body(a_ref, agg_ref, deg_ref, wl_ref, bl_ref, wr_ref, o_ref, *, relu):
    a = a_ref[...].astype(jnp.bfloat16)
    agg = agg_ref[...].astype(jnp.bfloat16)
    deg = deg_ref[...]
    acc = jnp.zeros((BT, D), jnp.float32)
    for d in range(MAX_DEG + 1):
        z = (jnp.dot(agg, wl_ref[d], preferred_element_type=jnp.float32)
             + jnp.dot(a, wr_ref[d], preferred_element_type=jnp.float32)
             + bl_ref[d])
        acc = acc + jnp.where(deg == float(d), z, 0.0)
    o_ref[...] = jnp.maximum(acc, 0.0) if relu else acc


def _mf_call(a, agg, degc, Wl, bl, Wr, relu):
    return pl.pallas_call(
        functools.partial(_mf_body, relu=relu),
        grid=(GRID,),
        in_specs=[
            pl.BlockSpec((BT, D), lambda i: (i, 0)),
            pl.BlockSpec((BT, D), lambda i: (i, 0)),
            pl.BlockSpec((BT, 1), lambda i: (i, 0)),
            pl.BlockSpec((MAX_DEG + 1, D, D), lambda i: (0, 0, 0)),
            pl.BlockSpec((MAX_DEG + 1, 1, D), lambda i: (0, 0, 0)),
            pl.BlockSpec((MAX_DEG + 1, D, D), lambda i: (0, 0, 0)),
        ],
        out_specs=pl.BlockSpec((BT, D), lambda i: (i, 0)),
        out_shape=jax.ShapeDtypeStruct((N_PAD, D), jnp.float32),
    )(a, agg, degc, Wl, bl.reshape(MAX_DEG + 1, 1, D), Wr)


def _pool_body(oh_ref, h_ref, w1_ref, b1_ref, w2_ref, b2_ref, y_ref, pacc):
    i = pl.program_id(0)

    @pl.when(i == 0)
    def _():
        pacc[...] = jnp.zeros((NUM_GRAPHS, D), jnp.float32)

    pacc[...] += jax.lax.dot_general(
        oh_ref[...], h_ref[...], (((0,), (0,)), ((), ())),
        preferred_element_type=jnp.float32)

    @pl.when(i == GRID - 1)
    def _():
        t = jnp.maximum(jnp.dot(pacc[...], w1_ref[...],
                                preferred_element_type=jnp.float32)
                        + b1_ref[...], 0.0)
        y_ref[...] = jnp.dot(t, w2_ref[...],
                             preferred_element_type=jnp.float32) + b2_ref[...]


def _pool_call(onehot, h, h1_W, h1_b, h2_W, h2_b):
    return pl.pallas_call(
        _pool_body,
        grid=(GRID,),
        in_specs=[
            pl.BlockSpec((BT, NUM_GRAPHS), lambda i: (i, 0)),
            pl.BlockSpec((BT, D), lambda i: (i, 0)),
            pl.BlockSpec((D, D), lambda i: (0, 0)),
            pl.BlockSpec((1, D), lambda i: (0, 0)),
            pl.BlockSpec((D, N_OUT), lambda i: (0, 0)),
            pl.BlockSpec((1, N_OUT), lambda i: (0, 0)),
        ],
        out_specs=pl.BlockSpec((NUM_GRAPHS, N_OUT), lambda i: (0, 0)),
        out_shape=jax.ShapeDtypeStruct((NUM_GRAPHS, N_OUT), jnp.float32),
        scratch_shapes=[pltpu.VMEM((NUM_GRAPHS, D), jnp.float32)],
    )(onehot, h, h1_W, h1_b.reshape(1, D), h2_W, h2_b.reshape(1, N_OUT))


def kernel(x, edge_index, edge_attr, batch, emb_W, emb_b, rgcn_W, rgcn_root,
           rgcn_b, mf_Wl, mf_bl, mf_Wr, h1_W, h1_b, h2_W, h2_b):
    src = edge_index[0].astype(jnp.int32)
    dst = edge_index[1].astype(jnp.int32)
    etype = jnp.argmax(edge_attr, axis=-1).astype(jnp.int32)
    key4 = dst * NUM_REL + etype
    merged = key4 | (src << 16)
    merged = jnp.pad(merged, (0, E_PAD - E), constant_values=PAD_KEY)

    hist = _hist_call(merged)
    hist = hist[:, :NB + 1]
    tot = jnp.sum(hist, axis=0)
    sizes16 = _ceil16(tot)
    boff = jnp.concatenate([jnp.zeros((1,), jnp.int32),
                            jnp.cumsum(sizes16)]).astype(jnp.int32)
    pt_excl = jnp.cumsum(hist, axis=0) - hist
    start_t = boff[None, :NB + 1] + pt_excl
    start_t = jnp.pad(start_t, ((0, 0), (0, 144 - (NB + 1))))

    widv = jnp.arange(NW, dtype=jnp.int32)
    cols = []
    for k in range(NUM_REL):
        b = k * NW + widv
        cols += [boff[b], tot[b]]
    for j in range(NUM_REL):
        b = widv * NUM_REL + j
        cols += [boff[b], tot[b]]
    ranges = jnp.stack(cols, axis=1).astype(jnp.int32)

    sedges = _scatter_call(merged, start_t.astype(jnp.int32))
    cnt = _cnt_call(sedges, ranges)

    cnt4 = cnt.reshape(N_PAD, NUM_REL).astype(jnp.float32)
    inv4c = (1.0 / jnp.maximum(cnt4, 1.0)).reshape(N_PAD, NUM_REL, 1)
    degc = jnp.minimum(jnp.sum(cnt4, axis=1),
                       float(MAX_DEG)).reshape(N_PAD, 1)

    xp = jnp.pad(x, ((0, N_PAD - N), (0, 0)))
    batchp = jnp.pad(batch.astype(jnp.int32), (0, N_PAD - N),
                     constant_values=NUM_GRAPHS)
    onehot = (batchp[:, None] == jnp.arange(NUM_GRAPHS)).astype(jnp.float32)

    a = _emb_call(xp, emb_W, emb_b)
    for blk in range(NUM_BLOCKS):
        buf4 = _rows_call(sedges, ranges, a, mf=False)
        buf4 = buf4.reshape(N_PAD, NUM_REL, D)
        a = _rgcn_call(a, buf4, inv4c,
                       rgcn_root[blk].astype(jnp.bfloat16),
                       rgcn_W[blk].astype(jnp.bfloat16), rgcn_b[blk])
        agg = _rows_call(sedges, ranges, a, mf=True)
        a = _mf_call(a, agg, degc, mf_Wl[blk].astype(jnp.bfloat16),
                     mf_bl[blk], mf_Wr[blk].astype(jnp.bfloat16),
                     relu=(blk < NUM_BLOCKS - 1))
    return _pool_call(onehot, a, h1_W, h1_b, h2_W, h2_b)


# speedup vs baseline: 1.0155x; 1.0155x over previous
import functools

import jax
import jax.numpy as jnp
from jax import lax
from jax.experimental import pallas as pl
from jax.experimental.pallas import tpu as pltpu
from jax.experimental.pallas import tpu_sc as plsc

N = 10000
E = 160000
D = 256
NUM_GRAPHS = 16
MAX_DEG = 10
NUM_REL = 4
NUM_BLOCKS = 2
N_OUT = 128

NC = 2
NS = 16
NW = NC * NS
L = 16

N_PAD = 10240
R4 = NUM_REL * N_PAD
BK = 320
NB = R4 // BK
E_PAD = 163840
SLICE = E_PAD // NW
SP_CAP = 170000
KE = 4096
PAD_KEY = 0xFFFF

_SC_PARAMS = pltpu.CompilerParams(needs_layout_passes=False)


def _mesh():
    return plsc.VectorSubcoreMesh(core_axis_name="c", subcore_axis_name="s")


def _wid():
    return lax.axis_index("s") * NC + lax.axis_index("c")


def _hist_body(ed_hbm, hist_hbm, ebuf, hacc):
    wid = _wid()
    zero16 = jnp.zeros((L,), jnp.int32)

    for i in range(144 // L):
        hacc[pl.ds(i * L, L)] = zero16
    pltpu.sync_copy(ed_hbm.at[pl.ds(wid * SLICE, SLICE)], ebuf)

    def hgrp(g, _):
        mv = ebuf[pl.ds(g * L, L)]
        kv = mv & 0xFFFF
        bv = jnp.minimum(kv // BK, NB)
        rc, lastm = plsc.scan_count(bv)
        plsc.addupdate_scatter(hacc, [bv], rc, mask=lastm)
        return 0

    lax.fori_loop(0, SLICE // L, hgrp, 0)
    pltpu.sync_copy(hacc, hist_hbm.at[wid])


def _hist_call(edges):
    kfn = pl.kernel(
        _hist_body,
        out_type=jax.ShapeDtypeStruct((NW, 144), jnp.int32),
        mesh=_mesh(),
        scratch_types=[
            pltpu.VMEM((SLICE,), jnp.int32),
            pltpu.VMEM((144,), jnp.int32),
        ],
        compiler_params=_SC_PARAMS,
    )
    return kfn(edges)


def _cnt_body(sed_hbm, rng_hbm, cnt_hbm, rbuf, ebuf, cacc):
    wid = _wid()
    kpt = R4 // NW
    lo = wid * kpt
    iota = lax.iota(jnp.int32, L)
    pltpu.sync_copy(rng_hbm.at[wid], rbuf)
    rv = rbuf[pl.ds(0, L)]
    zero16 = jnp.zeros((L,), jnp.int32)
    for i in range((kpt + L) // L):
        cacc[pl.ds(i * L, L)] = zero16

    for sub in range(NUM_REL):
        est = pl.multiple_of(rv[8 + 2 * sub], 16)
        ecnt = rv[9 + 2 * sub]
        nch = (ecnt + KE - 1) // KE

        def chunk_body(ch, _):
            off0 = ch * KE
            pltpu.sync_copy(sed_hbm.at[pl.ds(est + off0, KE)],
                            ebuf.at[pl.ds(0, KE)])
            rem_c = ecnt - off0
            ng = jnp.minimum((rem_c + L - 1) // L, KE // L)

            def cgrp(g, _):
                mv = ebuf[pl.ds(g * L, L)]
                kv = mv & 0xFFFF
                valid = iota < (rem_c - g * L)
                lidx = jnp.where(valid, kv - lo, kpt)
                rc, lastm = plsc.scan_count(lidx, mask=valid)
                plsc.addupdate_scatter(cacc, [lidx], rc, mask=lastm)
                return 0

            lax.fori_loop(0, ng, cgrp, 0)
            return 0

        lax.fori_loop(0, nch, chunk_body, 0)
    pltpu.sync_copy(cacc.at[pl.ds(0, kpt)], cnt_hbm.at[pl.ds(lo, kpt)])


def _cnt_call(sedges, ranges):
    kfn = pl.kernel(
        _cnt_body,
        out_type=jax.ShapeDtypeStruct((R4,), jnp.int32),
        mesh=_mesh(),
        scratch_types=[
            pltpu.VMEM((L,), jnp.int32),
            pltpu.VMEM((KE + L,), jnp.int32),
            pltpu.VMEM((R4 // NW + L,), jnp.int32),
        ],
        compiler_params=_SC_PARAMS,
    )
    return kfn(sedges, ranges)


def _scatter_body(ed_hbm, start_hbm, out_hbm, ebuf, ctr, drain, sem):
    wid = _wid()
    pltpu.sync_copy(ed_hbm.at[pl.ds(wid * SLICE, SLICE)], ebuf)
    pltpu.sync_copy(start_hbm.at[wid], ctr)

    def grp(g, _):
        mv = ebuf[pl.ds(g * L, L)]
        kv = mv & 0xFFFF
        bv = jnp.minimum(kv // BK, NB)
        rc, lastm = plsc.scan_count(bv)
        base = plsc.load_gather(ctr, [bv])
        posv = base + rc - 1
        plsc.addupdate_scatter(ctr, [bv], rc, mask=lastm)
        pltpu.async_copy(ebuf.at[pl.ds(g * L, L)], out_hbm.at[posv], sem)
        return 0

    lax.fori_loop(0, SLICE // L, grp, 0)

    def dr(g, _):
        pltpu.make_async_copy(ed_hbm.at[pl.ds(0, L)], drain, sem).wait()
        return 0

    lax.fori_loop(0, SLICE // L, dr, 0)


def _scatter_call(edges, start_t):
    kfn = pl.kernel(
        _scatter_body,
        out_type=jax.ShapeDtypeStruct((SP_CAP,), jnp.int32),
        mesh=_mesh(),
        scratch_types=[
            pltpu.VMEM((SLICE,), jnp.int32),
            pltpu.VMEM((144,), jnp.int32),
            pltpu.VMEM((L,), jnp.int32),
            pltpu.SemaphoreType.DMA,
        ],
        compiler_params=_SC_PARAMS,
    )
    return kfn(edges, start_t)


def _rows_body(sed_hbm, rng_hbm, h_hbm, out_hbm, rbuf, ebuf, stg0, stg1, acc,
               sem0, sem1, *, mf):
    wid = _wid()
    pltpu.sync_copy(rng_hbm.at[wid], rbuf)
    rv = rbuf[pl.ds(0, L)]
    iota = lax.iota(jnp.int32, L)
    zero16 = jnp.zeros((L,), jnp.float32)

    n_rounds = 1 if mf else NUM_REL
    n_sub = NUM_REL if mf else 1
    for rnd in range(n_rounds):
        def zrow(i, _):
            for kk in range(D // L):
                acc[i, pl.ds(kk * L, L)] = zero16
            return 0

        lax.fori_loop(0, BK, zrow, 0)

        for sub in range(n_sub):
            base_w = (8 + 2 * sub) if mf else (2 * rnd)
            est = pl.multiple_of(rv[base_w], 16)
            ecnt = rv[base_w + 1]
            nch = (ecnt + KE - 1) // KE

            def lane_meta(rem_c, g):
                off = g * L
                mv = ebuf[pl.ds(off, L)]
                kv = mv & 0xFFFF
                sv = mv >> 16
                valid = iota < (rem_c - off)
                sv = jnp.where(valid, sv, 0)
                if mf:
                    lidx = (kv >> 2) - wid * BK
                else:
                    lidx = kv - (rnd * NW + wid) * BK
                lidx = jnp.where(valid, lidx, BK)
                return sv, lidx

            def gather(rem_c, g, stg, sem):
                sv, _ = lane_meta(rem_c, g)
                pltpu.async_copy(h_hbm.at[sv], stg, sem)

            def process(rem_c, g, stg, sem):
                _, lidx = lane_meta(rem_c, g)
                pltpu.make_async_copy(
                    h_hbm.at[pl.ds(0, L), :], stg, sem).wait()
                for i in range(L):
                    li = lidx[i]

                    @plsc.parallel_loop(0, D, step=L, unroll=4)
                    def _(kk):
                        plsc.addupdate(acc.at[li, pl.ds(kk, L)],
                                       stg[i, pl.ds(kk, L)])

            def chunk_body(ch, _):
                off0 = ch * KE
                pltpu.sync_copy(sed_hbm.at[pl.ds(est + off0, KE)],
                                ebuf.at[pl.ds(0, KE)])
                rem_c = ecnt - off0
                ng = jnp.minimum((rem_c + L - 1) // L, KE // L)
                ngp = (ng + 1) // 2
                gather(rem_c, 0, stg0, sem0)

                def pair(p, _):
                    g0 = 2 * p
                    gather(rem_c, g0 + 1, stg1, sem1)
                    process(rem_c, g0, stg0, sem0)
                    gather(rem_c, g0 + 2, stg0, sem0)
                    process(rem_c, g0 + 1, stg1, sem1)
                    return 0

                lax.fori_loop(0, ngp, pair, 0)
                pltpu.make_async_copy(
                    h_hbm.at[pl.ds(0, L), :], stg0, sem0).wait()
                return 0

            lax.fori_loop(0, nch, chunk_body, 0)

        obase = (wid if mf else rnd * NW + wid) * BK
        pltpu.sync_copy(acc.at[pl.ds(0, BK), :], out_hbm.at[pl.ds(obase, BK), :])


def _rows_call(sedges, ranges, h, mf):
    rows = N_PAD if mf else R4
    kfn = pl.kernel(
        functools.partial(_rows_body, mf=mf),
        out_type=jax.ShapeDtypeStruct((rows, D), jnp.float32),
        mesh=_mesh(),
        scratch_types=[
            pltpu.VMEM((L,), jnp.int32),
            pltpu.VMEM((KE + 2 * L,), jnp.int32),
            pltpu.VMEM((L, D), jnp.float32),
            pltpu.VMEM((L, D), jnp.float32),
            pltpu.VMEM((BK + L, D), jnp.float32),
            pltpu.SemaphoreType.DMA,
            pltpu.SemaphoreType.DMA,
        ],
        compiler_params=_SC_PARAMS,
    )
    return kfn(sedges, ranges, h)


def _ceil16(x):
    return ((x + 15) // 16) * 16


BT = 256
GRID = N_PAD // BT


def _emb_body(x_ref, w_ref, b_ref, o_ref):
    o_ref[...] = jnp.maximum(
        jnp.dot(x_ref[...], w_ref[...],
                preferred_element_type=jnp.float32) + b_ref[...], 0.0)


def _emb_call(xp, emb_W, emb_b):
    return pl.pallas_call(
        _emb_body,
        grid=(GRID,),
        in_specs=[
            pl.BlockSpec((BT, D), lambda i: (i, 0)),
            pl.BlockSpec((D, D), lambda i: (0, 0)),
            pl.BlockSpec((1, D), lambda i: (0, 0)),
        ],
        out_specs=pl.BlockSpec((BT, D), lambda i: (i, 0)),
        out_shape=jax.ShapeDtypeStruct((N_PAD, D), jnp.float32),
    )(xp, emb_W, emb_b.reshape(1, D))


def _rgcn_body(a_ref, buf_ref, inv_ref, root_ref, w_ref, b_ref, o_ref):
    a = a_ref[...].astype(jnp.bfloat16)
    acc = jnp.dot(a, root_ref[...], preferred_element_type=jnp.float32)
    for r in range(NUM_REL):
        mean = (buf_ref[:, r, :] * inv_ref[:, r, :]).astype(jnp.bfloat16)
        acc = acc + jnp.dot(mean, w_ref[r], preferred_element_type=jnp.float32)
    o_ref[...] = jnp.maximum(acc + b_ref[...], 0.0)


def _rgcn_call(a, buf4, inv4c, root, W, b):
    return pl.pallas_call(
        _rgcn_body,
        grid=(GRID,),
        in_specs=[
            pl.BlockSpec((BT, D), lambda i: (i, 0)),
            pl.BlockSpec((BT, NUM_REL, D), lambda i: (i, 0, 0)),
            pl.BlockSpec((BT, NUM_REL, 1), lambda i: (i, 0, 0)),
            pl.BlockSpec((D, D), lambda i: (0, 0)),
            pl.BlockSpec((NUM_REL, D, D), lambda i: (0, 0, 0)),
            pl.BlockSpec((1, D), lambda i: (0, 0)),
        ],
        out_specs=pl.BlockSpec((BT, D), lambda i: (i, 0)),
        out_shape=jax.ShapeDtypeStruct((N_PAD, D), jnp.float32),
    )(a, buf4, inv4c, root, W, b.reshape(1, D))


def _mf_body(a_ref, agg_ref, deg_ref, wl_ref, bl_ref, wr_ref, o_ref, *, relu):
    a = a_ref[...].astype(jnp.bfloat16)
    agg = agg_ref[...].astype(jnp.bfloat16)
    deg = deg_ref[...]
    acc = jnp.zeros((BT, D), jnp.float32)
    for d in range(MAX_DEG + 1):
        z = (jnp.dot(agg, wl_ref[d], preferred_element_type=jnp.float32)
             + jnp.dot(a, wr_ref[d], preferred_element_type=jnp.float32)
             + bl_ref[d])
        acc = acc + jnp.where(deg == float(d), z, 0.0)
    o_ref[...] = jnp.maximum(acc, 0.0) if relu else acc


def _mf_call(a, agg, degc, Wl, bl, Wr, relu):
    return pl.pallas_call(
        functools.partial(_mf_body, relu=relu),
        grid=(GRID,),
        in_specs=[
            pl.BlockSpec((BT, D), lambda i: (i, 0)),
            pl.BlockSpec((BT, D), lambda i: (i, 0)),
            pl.BlockSpec((BT, 1), lambda i: (i, 0)),
            pl.BlockSpec((MAX_DEG + 1, D, D), lambda i: (0, 0, 0)),
            pl.BlockSpec((MAX_DEG + 1, 1, D), lambda i: (0, 0, 0)),
            pl.BlockSpec((MAX_DEG + 1, D, D), lambda i: (0, 0, 0)),
        ],
        out_specs=pl.BlockSpec((BT, D), lambda i: (i, 0)),
        out_shape=jax.ShapeDtypeStruct((N_PAD, D), jnp.float32),
    )(a, agg, degc, Wl, bl.reshape(MAX_DEG + 1, 1, D), Wr)


def _pool_body(oh_ref, h_ref, w1_ref, b1_ref, w2_ref, b2_ref, y_ref, pacc):
    i = pl.program_id(0)

    @pl.when(i == 0)
    def _():
        pacc[...] = jnp.zeros((NUM_GRAPHS, D), jnp.float32)

    pacc[...] += jax.lax.dot_general(
        oh_ref[...], h_ref[...], (((0,), (0,)), ((), ())),
        preferred_element_type=jnp.float32)

    @pl.when(i == GRID - 1)
    def _():
        t = jnp.maximum(jnp.dot(pacc[...], w1_ref[...],
                                preferred_element_type=jnp.float32)
                        + b1_ref[...], 0.0)
        y_ref[...] = jnp.dot(t, w2_ref[...],
                             preferred_element_type=jnp.float32) + b2_ref[...]


def _pool_call(onehot, h, h1_W, h1_b, h2_W, h2_b):
    return pl.pallas_call(
        _pool_body,
        grid=(GRID,),
        in_specs=[
            pl.BlockSpec((BT, NUM_GRAPHS), lambda i: (i, 0)),
            pl.BlockSpec((BT, D), lambda i: (i, 0)),
            pl.BlockSpec((D, D), lambda i: (0, 0)),
            pl.BlockSpec((1, D), lambda i: (0, 0)),
            pl.BlockSpec((D, N_OUT), lambda i: (0, 0)),
            pl.BlockSpec((1, N_OUT), lambda i: (0, 0)),
        ],
        out_specs=pl.BlockSpec((NUM_GRAPHS, N_OUT), lambda i: (0, 0)),
        out_shape=jax.ShapeDtypeStruct((NUM_GRAPHS, N_OUT), jnp.float32),
        scratch_shapes=[pltpu.VMEM((NUM_GRAPHS, D), jnp.float32)],
    )(onehot, h, h1_W, h1_b.reshape(1, D), h2_W, h2_b.reshape(1, N_OUT))


def kernel(x, edge_index, edge_attr, batch, emb_W, emb_b, rgcn_W, rgcn_root,
           rgcn_b, mf_Wl, mf_bl, mf_Wr, h1_W, h1_b, h2_W, h2_b):
    src = edge_index[0].astype(jnp.int32)
    dst = edge_index[1].astype(jnp.int32)
    etype = jnp.argmax(edge_attr, axis=-1).astype(jnp.int32)
    key4 = dst * NUM_REL + etype
    merged = key4 | (src << 16)
    merged = jnp.pad(merged, (0, E_PAD - E), constant_values=PAD_KEY)

    hist = _hist_call(merged)
    hist = hist[:, :NB + 1]
    tot = jnp.sum(hist, axis=0)
    sizes16 = _ceil16(tot)
    boff = jnp.concatenate([jnp.zeros((1,), jnp.int32),
                            jnp.cumsum(sizes16)]).astype(jnp.int32)
    pt_excl = jnp.cumsum(hist, axis=0) - hist
    start_t = boff[None, :NB + 1] + pt_excl
    start_t = jnp.pad(start_t, ((0, 0), (0, 144 - (NB + 1))))

    widv = jnp.arange(NW, dtype=jnp.int32)
    cols = []
    for k in range(NUM_REL):
        b = k * NW + widv
        cols += [boff[b], tot[b]]
    for j in range(NUM_REL):
        b = widv * NUM_REL + j
        cols += [boff[b], tot[b]]
    ranges = jnp.stack(cols, axis=1).astype(jnp.int32)

    sedges = _scatter_call(merged, start_t.astype(jnp.int32))
    cnt = _cnt_call(sedges, ranges)

    cnt4 = cnt.reshape(N_PAD, NUM_REL).astype(jnp.float32)
    inv4c = (1.0 / jnp.maximum(cnt4, 1.0)).reshape(N_PAD, NUM_REL, 1)
    degc = jnp.minimum(jnp.sum(cnt4, axis=1),
                       float(MAX_DEG)).reshape(N_PAD, 1)

    xp = jnp.pad(x, ((0, N_PAD - N), (0, 0)))
    batchp = jnp.pad(batch.astype(jnp.int32), (0, N_PAD - N),
                     constant_values=NUM_GRAPHS)
    onehot = (batchp[:, None] == jnp.arange(NUM_GRAPHS)).astype(jnp.float32)

    a = _emb_call(xp, emb_W, emb_b)
    for blk in range(NUM_BLOCKS):
        buf4 = _rows_call(sedges, ranges, a, mf=False)
        buf4 = buf4.reshape(N_PAD, NUM_REL, D)
        a = _rgcn_call(a, buf4, inv4c,
                       rgcn_root[blk].astype(jnp.bfloat16),
                       rgcn_W[blk].astype(jnp.bfloat16), rgcn_b[blk])
        agg = _rows_call(sedges, ranges, a, mf=True)
        a = _mf_call(a, agg, degc, mf_Wl[blk].astype(jnp.bfloat16),
                     mf_bl[blk], mf_Wr[blk].astype(jnp.bfloat16),
                     relu=(blk < NUM_BLOCKS - 1))
    return _pool_call(onehot, a, h1_W, h1_b, h2_W, h2_b)
